# baseline (device time: 34845 ns/iter reference)
import jax
import jax.numpy as jnp
from jax import lax
from jax.experimental import pallas as pl
from jax.experimental.pallas import tpu as pltpu

C = 32


def kernel(x):
    m, n = x.shape
    half = m // 2
    rows = half // C

    def body(x_ref, out_ref, xbuf, comm1, comm2,
             send1, recv1, send2, recv2, in_sem):
        my_x = lax.axis_index("x")
        my_y = lax.axis_index("y")
        x_partner = (1 - my_x, my_y)
        y_partner = (my_x, 1 - my_y)
        base = my_y * half
        obase = (1 - my_y) * half

        local_in = pltpu.make_async_copy(x_ref, xbuf, in_sem)
        local_in.start()

        barrier_sem = pltpu.get_barrier_semaphore()
        for nbr in (x_partner, y_partner):
            pl.semaphore_signal(
                barrier_sem, inc=1,
                device_id=nbr, device_id_type=pl.DeviceIdType.MESH,
            )
        pl.semaphore_wait(barrier_sem, 2)

        rdma1 = []
        for c in range(C):
            r = pltpu.make_async_remote_copy(
                src_ref=x_ref.at[pl.ds(base + c * rows, rows), :],
                dst_ref=comm1.at[pl.ds(c * rows, rows), :],
                send_sem=send1.at[c],
                recv_sem=recv1.at[c],
                device_id=x_partner,
                device_id_type=pl.DeviceIdType.MESH,
            )
            r.start()
            rdma1.append(r)

        local_in.wait()

        rdma2 = []
        for c in range(C):
            rdma1[c].wait_recv()
            csl = pl.ds(c * rows, rows)
            r = pltpu.make_async_remote_copy(
                src_ref=comm1.at[csl, :],
                dst_ref=comm2.at[csl, :],
                send_sem=send2.at[c],
                recv_sem=recv2.at[c],
                device_id=y_partner,
                device_id_type=pl.DeviceIdType.MESH,
            )
            r.start()
            rdma2.append(r)
            sl = pl.ds(base + c * rows, rows)
            out_ref[sl, :] = xbuf[sl, :] + comm1[csl, :]

        for c in range(C):
            rdma2[c].wait_recv()
            csl = pl.ds(c * rows, rows)
            sl = pl.ds(obase + c * rows, rows)
            out_ref[sl, :] = xbuf[sl, :] + comm2[csl, :]
        for c in range(C):
            rdma1[c].wait_send()
            rdma2[c].wait_send()

    return pl.pallas_call(
        body,
        out_shape=jax.ShapeDtypeStruct((m, n), x.dtype),
        in_specs=[pl.BlockSpec(memory_space=pl.ANY)],
        out_specs=pl.BlockSpec(memory_space=pltpu.VMEM),
        scratch_shapes=[
            pltpu.VMEM((m, n), x.dtype),
            pltpu.VMEM((half, n), x.dtype),
            pltpu.VMEM((half, n), x.dtype),
            pltpu.SemaphoreType.DMA((C,)),
            pltpu.SemaphoreType.DMA((C,)),
            pltpu.SemaphoreType.DMA((C,)),
            pltpu.SemaphoreType.DMA((C,)),
            pltpu.SemaphoreType.DMA,
        ],
        compiler_params=pltpu.CompilerParams(collective_id=0),
    )(x)


# device time: 34563 ns/iter; 1.0082x vs baseline; 1.0082x over previous
import jax
import jax.numpy as jnp
from jax import lax
from jax.experimental import pallas as pl
from jax.experimental.pallas import tpu as pltpu

C = 32


def kernel(x):
    m, n = x.shape
    half = m // 2
    rows = half // C

    def body(x_ref, out_ref, comm1, comm2, send1, recv1, send2, recv2):
        my_x = lax.axis_index("x")
        my_y = lax.axis_index("y")
        x_partner = (1 - my_x, my_y)
        y_partner = (my_x, 1 - my_y)
        base = my_y * half
        obase = (1 - my_y) * half

        barrier_sem = pltpu.get_barrier_semaphore()
        for nbr in (x_partner, y_partner):
            pl.semaphore_signal(
                barrier_sem, inc=1,
                device_id=nbr, device_id_type=pl.DeviceIdType.MESH,
            )
        pl.semaphore_wait(barrier_sem, 2)

        rdma1 = []
        for c in range(C):
            r = pltpu.make_async_remote_copy(
                src_ref=x_ref.at[pl.ds(base + c * rows, rows), :],
                dst_ref=comm1.at[pl.ds(c * rows, rows), :],
                send_sem=send1.at[c],
                recv_sem=recv1.at[c],
                device_id=x_partner,
                device_id_type=pl.DeviceIdType.MESH,
            )
            r.start()
            rdma1.append(r)

        rdma2 = []
        for c in range(C):
            rdma1[c].wait_recv()
            csl = pl.ds(c * rows, rows)
            r = pltpu.make_async_remote_copy(
                src_ref=comm1.at[csl, :],
                dst_ref=comm2.at[csl, :],
                send_sem=send2.at[c],
                recv_sem=recv2.at[c],
                device_id=y_partner,
                device_id_type=pl.DeviceIdType.MESH,
            )
            r.start()
            rdma2.append(r)
            sl = pl.ds(base + c * rows, rows)
            out_ref[sl, :] = x_ref[sl, :] + comm1[csl, :]

        for c in range(C):
            rdma2[c].wait_recv()
            csl = pl.ds(c * rows, rows)
            sl = pl.ds(obase + c * rows, rows)
            out_ref[sl, :] = x_ref[sl, :] + comm2[csl, :]
        for c in range(C):
            rdma1[c].wait_send()
            rdma2[c].wait_send()

    return pl.pallas_call(
        body,
        out_shape=jax.ShapeDtypeStruct((m, n), x.dtype),
        in_specs=[pl.BlockSpec(memory_space=pltpu.VMEM)],
        out_specs=pl.BlockSpec(memory_space=pltpu.VMEM),
        scratch_shapes=[
            pltpu.VMEM((half, n), x.dtype),
            pltpu.VMEM((half, n), x.dtype),
            pltpu.SemaphoreType.DMA((C,)),
            pltpu.SemaphoreType.DMA((C,)),
            pltpu.SemaphoreType.DMA((C,)),
            pltpu.SemaphoreType.DMA((C,)),
        ],
        compiler_params=pltpu.CompilerParams(collective_id=0),
    )(x)
